# baseline (speedup 1.0000x reference)
import functools

import jax
import jax.numpy as jnp
import numpy as np
from jax import lax
from jax.experimental import pallas as pl
from jax.experimental.pallas import tpu as pltpu
from jax.experimental.pallas import tpu_sc as plsc

N = 10000
NPAD = 10240
HID = 128
LAYERS = 4
HEADS = 8
BASES = 4
F = HID // HEADS
DESC = 200
NGRAPH = 128

NC = 2
NS = 16
NW = NC * NS
CH = 128


def _sc_degree_kernel(epw_chunks):
    mesh = plsc.VectorSubcoreMesh(core_axis_name="c", subcore_axis_name="s")
    rows_per_s = NPAD // NS

    @functools.partial(
        pl.kernel,
        mesh=mesh,
        out_type=jax.ShapeDtypeStruct((NC, NPAD), jnp.float32),
        scratch_types=[
            pltpu.VMEM((epw_chunks, CH), jnp.int32),
            pltpu.VMEM((CH,), jnp.float32),
            pltpu.VMEM((rows_per_s,), jnp.float32),
            pltpu.VMEM_SHARED((NPAD,), jnp.float32),
        ],
    )
    def k(dst_hbm, out_hbm, idx_v, ones_v, zeros_v, cnt_sh):
        c = lax.axis_index("c")
        s = lax.axis_index("s")
        wid = s * NC + c
        for i in range(CH // 16):
            ones_v[pl.ds(i * 16, 16)] = jnp.ones((16,), jnp.float32)

        def zbody(i, carry):
            zeros_v[pl.ds(i * 16, 16)] = jnp.zeros((16,), jnp.float32)
            return carry

        lax.fori_loop(0, rows_per_s // 16, zbody, 0)
        pltpu.sync_copy(zeros_v, cnt_sh.at[pl.ds(s * rows_per_s, rows_per_s)])
        pltpu.sync_copy(dst_hbm.at[wid], idx_v)
        plsc.subcore_barrier()

        def body(j, carry):
            pltpu.sync_copy(ones_v, cnt_sh.at[idx_v.at[j]], add=True)
            return carry

        lax.fori_loop(0, epw_chunks, body, 0)
        plsc.subcore_barrier()
        pltpu.sync_copy(
            cnt_sh.at[pl.ds(s * rows_per_s, rows_per_s)],
            out_hbm.at[c, pl.ds(s * rows_per_s, rows_per_s)],
        )

    return k


def _sc_layer_kernel(epw_chunks):
    mesh = plsc.VectorSubcoreMesh(core_axis_name="c", subcore_axis_name="s")
    rows_per_s = NPAD // NS

    @functools.partial(
        pl.kernel,
        mesh=mesh,
        out_type=jax.ShapeDtypeStruct((NC, NPAD, BASES * F), jnp.float32),
        compiler_params=pltpu.CompilerParams(use_tc_tiling_on_sc=False),
        scratch_types=[
            pltpu.VMEM((epw_chunks, CH), jnp.int32),
            pltpu.VMEM((epw_chunks, CH), jnp.int32),
            pltpu.VMEM((CH, BASES * F), jnp.float32),
            pltpu.VMEM((CH, BASES * F), jnp.float32),
            pltpu.VMEM_SHARED((NPAD, BASES * F), jnp.float32),
            pltpu.SemaphoreType.DMA,
        ],
    )
    def k(src_hbm, dst_hbm, bases_hbm, out_hbm, sidx_v, didx_v, zeros_v,
          rows_v, agg_sh, sem):
        c = lax.axis_index("c")
        s = lax.axis_index("s")
        wid = s * NC + c

        def zb(i, carry):
            def zb2(j, carry2):
                zeros_v[i, pl.ds(j * 16, 16)] = jnp.zeros((16,), jnp.float32)
                return carry2
            return lax.fori_loop(0, (BASES * F) // 16, zb2, carry)

        lax.fori_loop(0, CH, zb, 0)
        def zcopy(i, carry):
            pltpu.sync_copy(zeros_v, agg_sh.at[pl.ds(s * rows_per_s + i * CH, CH)])
            return carry

        lax.fori_loop(0, rows_per_s // CH, zcopy, 0)
        pltpu.sync_copy(src_hbm.at[wid], sidx_v)
        pltpu.sync_copy(dst_hbm.at[wid], didx_v)
        plsc.subcore_barrier()

        def body(j, carry):
            pltpu.async_copy(bases_hbm.at[sidx_v.at[j]], rows_v, sem).wait()
            pltpu.sync_copy(rows_v, agg_sh.at[didx_v.at[j]], add=True)
            return carry

        lax.fori_loop(0, epw_chunks, body, 0)
        plsc.subcore_barrier()
        pltpu.sync_copy(
            agg_sh.at[pl.ds(s * rows_per_s, rows_per_s)],
            out_hbm.at[c, pl.ds(s * rows_per_s, rows_per_s)],
        )

    return k


def _pad_edges(idx, total):
    pad = total - idx.shape[0]
    pad_rows = jnp.asarray(N + (np.arange(pad) % (NPAD - N)), jnp.int32)
    return jnp.concatenate([idx, pad_rows])


def _bn(x, g, b, eps=1e-5):
    mu = jnp.mean(x, axis=0, keepdims=True)
    var = jnp.mean((x - mu) * (x - mu), axis=0, keepdims=True)
    return (x - mu) * lax.rsqrt(var + eps) * g + b


def _dot(a, b):
    return jax.lax.dot_general(
        a, b, (((1,), (0,)), ((), ())),
        preferred_element_type=jnp.float32)


def _dotT(a, b):
    return jax.lax.dot_general(
        a, b, (((0,), (0,)), ((), ())),
        preferred_element_type=jnp.float32)


_S_EXPAND = np.zeros((BASES, BASES * HEADS, HID), np.float32)
_T_EXPAND = np.zeros((BASES, BASES * F, HID), np.float32)
for _b in range(BASES):
    for _h in range(HEADS):
        for _f in range(F):
            _S_EXPAND[_b, _h * BASES + _b, _h * F + _f] = 1.0
            _T_EXPAND[_b, _b * F + _f, _h * F + _f] = 1.0


RB = 2000
NBLK = N // RB


def _blk(cols):
    return pl.BlockSpec((RB, cols), lambda i: (i, 0))


def _full(shape):
    nd = len(shape)
    return pl.BlockSpec(shape, lambda i: (0,) * nd)


def _tc_embed_a(xp, w1p, b1):
    def body(x_r, w1_r, b1_r, t_r, ps_r, pss_r):
        t = _dot(x_r[...], w1_r[...]) + b1_r[...]
        t_r[...] = t
        mb = jnp.sum(t, axis=0, keepdims=True) / RB
        d = t - mb
        ps_r[...] = mb.reshape(1, 1, HID)
        pss_r[...] = jnp.sum(d * d, axis=0, keepdims=True).reshape(1, 1, HID)

    return pl.pallas_call(
        body,
        grid=(NBLK,),
        in_specs=[_blk(32), _full((32, HID)), _full((1, HID))],
        out_specs=[_blk(HID), pl.BlockSpec((1, 1, HID), lambda i: (i, 0, 0)),
                   pl.BlockSpec((1, 1, HID), lambda i: (i, 0, 0))],
        out_shape=[
            jax.ShapeDtypeStruct((N, HID), jnp.float32),
            jax.ShapeDtypeStruct((NBLK, 1, HID), jnp.float32),
            jax.ShapeDtypeStruct((NBLK, 1, HID), jnp.float32),
        ],
    )(xp, w1p, b1)


def _finalize_stats(ps, pss, eps=1e-5):
    mu = jnp.sum(ps[...], axis=0) / NBLK
    dm = ps[...] - mu
    var = (jnp.sum(pss[...], axis=0) + RB * jnp.sum(dm * dm, axis=0)) / N
    return mu, lax.rsqrt(var + eps)


def _tc_embed_b(t, ps, pss, g1, be1, d0, d1, wb, wc, bc):
    def body(t_r, ps_r, pss_r, g_r, be_r, d0_r, d1_r, wb_r, wc_r, bc_r,
             h_r, dinv_r, bases_r, wt_r):
        mu, rstd = _finalize_stats(ps_r, pss_r)
        h = jax.nn.relu((t_r[...] - mu) * rstd * g_r[...] + be_r[...])
        h_r[...] = h
        dinv = lax.rsqrt(1.0 + d0_r[...] + d1_r[...])
        dinv_r[...] = dinv
        bases_r[...] = dinv * _dot(h, wb_r[...])
        wt_r[...] = _dot(h, wc_r[...]) + bc_r[...]

    return pl.pallas_call(
        body,
        grid=(NBLK,),
        in_specs=[_blk(HID), _full((NBLK, 1, HID)), _full((NBLK, 1, HID)),
                  _full((1, HID)), _full((1, HID)), _blk(1), _blk(1),
                  _full((HID, BASES * F)), _full((HID, HEADS * BASES)),
                  _full((1, HEADS * BASES))],
        out_specs=[_blk(HID), _blk(1), _blk(BASES * F), _blk(HEADS * BASES)],
        out_shape=[
            jax.ShapeDtypeStruct((N, HID), jnp.float32),
            jax.ShapeDtypeStruct((N, 1), jnp.float32),
            jax.ShapeDtypeStruct((N, BASES * F), jnp.float32),
            jax.ShapeDtypeStruct((N, HEADS * BASES), jnp.float32),
        ],
    )(t, ps, pss, g1, be1, d0, d1, wb, wc, bc)


def _tc_tail_a(dinv, wt, p0, p1, bias, s_e, t_e):
    def body(dinv_r, wt_r, p0_r, p1_r, bias_r, s_r, t_r, o_r, ps_r, pss_r):
        agg = dinv_r[...] * (p0_r[...] + p1_r[...])
        wt = wt_r[...]
        cols = []
        for h in range(HEADS):
            acc = None
            for b in range(BASES):
                c = h * BASES + b
                term = wt[:, c:c + 1] * agg[:, b * F:(b + 1) * F]
                acc = term if acc is None else acc + term
            cols.append(acc)
        o = jnp.concatenate(cols, axis=1) + bias_r[...]
        o_r[...] = o
        mb = jnp.sum(o, axis=0, keepdims=True) / RB
        d = o - mb
        ps_r[...] = mb.reshape(1, 1, HID)
        pss_r[...] = jnp.sum(d * d, axis=0, keepdims=True).reshape(1, 1, HID)

    return pl.pallas_call(
        body,
        grid=(NBLK,),
        in_specs=[_blk(1), _blk(HEADS * BASES), _blk(BASES * F),
                  _blk(BASES * F), _full((1, HID)),
                  _full((BASES, HEADS * BASES, HID)),
                  _full((BASES, BASES * F, HID))],
        out_specs=[_blk(HID), pl.BlockSpec((1, 1, HID), lambda i: (i, 0, 0)),
                   pl.BlockSpec((1, 1, HID), lambda i: (i, 0, 0))],
        out_shape=[
            jax.ShapeDtypeStruct((N, HID), jnp.float32),
            jax.ShapeDtypeStruct((NBLK, 1, HID), jnp.float32),
            jax.ShapeDtypeStruct((NBLK, 1, HID), jnp.float32),
        ],
    )(dinv, wt, p0, p1, bias, s_e, t_e)


def _tc_tail_b(h, o, ps, pss, g, be, dinv, wb, wc, bc):
    def body(h_r, o_r, ps_r, pss_r, g_r, be_r, dinv_r, wb_r, wc_r, bc_r,
             hn_r, bases_r, wt_r):
        mu, rstd = _finalize_stats(ps_r, pss_r)
        hn = h_r[...] + jax.nn.relu((o_r[...] - mu) * rstd * g_r[...]
                                    + be_r[...])
        hn_r[...] = hn
        bases_r[...] = dinv_r[...] * _dot(hn, wb_r[...])
        wt_r[...] = _dot(hn, wc_r[...]) + bc_r[...]

    return pl.pallas_call(
        body,
        grid=(NBLK,),
        in_specs=[_blk(HID), _blk(HID), _full((NBLK, 1, HID)),
                  _full((NBLK, 1, HID)), _full((1, HID)), _full((1, HID)),
                  _blk(1), _full((HID, BASES * F)),
                  _full((HID, HEADS * BASES)), _full((1, HEADS * BASES))],
        out_specs=[_blk(HID), _blk(BASES * F), _blk(HEADS * BASES)],
        out_shape=[
            jax.ShapeDtypeStruct((N, HID), jnp.float32),
            jax.ShapeDtypeStruct((N, BASES * F), jnp.float32),
            jax.ShapeDtypeStruct((N, HEADS * BASES), jnp.float32),
        ],
    )(h, o, ps, pss, g, be, dinv, wb, wc, bc)


def _tc_pool(hn, batch_col):
    def body(h_r, batch_r, ssum_r, cnt_r):
        i = pl.program_id(0)
        gids = jax.lax.broadcasted_iota(jnp.int32, (1, NGRAPH), 1)
        onehot = (batch_r[...] == gids).astype(jnp.float32)

        @pl.when(i == 0)
        def _():
            ssum_r[...] = jnp.zeros((NGRAPH, HID), jnp.float32)
            cnt_r[...] = jnp.zeros((NGRAPH, 1), jnp.float32)

        ssum_r[...] += _dotT(onehot, h_r[...])
        cnt_r[...] += _dotT(onehot, jnp.ones((RB, 1), jnp.float32))

    return pl.pallas_call(
        body,
        grid=(NBLK,),
        in_specs=[_blk(HID), _blk(1)],
        out_specs=[_full((NGRAPH, HID)), _full((NGRAPH, 1))],
        out_shape=[
            jax.ShapeDtypeStruct((NGRAPH, HID), jnp.float32),
            jax.ShapeDtypeStruct((NGRAPH, 1), jnp.float32),
        ],
    )(hn, batch_col)


def _tc_head(ssum, cnt, descriptors, Wm1, gm1, bm1, Wm2, gm2, bm2, W2m,
             W2d, b2, g3, be3, Wout, bout):
    def body(ssum_r, cnt_r, desc_r, wm1_r, gm1_r, bm1_r, wm2_r, gm2_r,
             bm2_r, w2m_r, w2d_r, b2_r, g3_r, be3_r, wout_r, bout_r, out_r):
        pooled = ssum_r[...] * (1.0 / jnp.maximum(cnt_r[...], 1.0))
        m = jax.nn.relu(_bn(_dot(pooled, wm1_r[...]), gm1_r[...], bm1_r[...]))
        m = jax.nn.relu(_bn(_dot(m, wm2_r[...]), gm2_r[...], bm2_r[...]))
        z = jax.nn.relu(_dot(m, w2m_r[...]) + _dot(desc_r[...], w2d_r[...])
                        + b2_r[...])
        z = _bn(z, g3_r[...], be3_r[...])
        out_r[...] = _dot(z, wout_r[...]) + bout_r[...]

    return pl.pallas_call(
        body,
        out_shape=jax.ShapeDtypeStruct((NGRAPH, 1), jnp.float32),
    )(ssum, cnt, descriptors, Wm1, gm1, bm1, Wm2, gm2, bm2, W2m, W2d,
      b2, g3, be3, Wout, bout)


def kernel(x, edge_index, batch, descriptors, W1, b1, g1, be1, convWb, convWc,
           convbc, convbias, convg, convbe, Wm1, gm1, bm1, Wm2, gm2, bm2, W2,
           b2, g3, be3, Wout, bout):
    n = N
    E = edge_index.shape[1]
    epd = ((E + NW * CH - 1) // (NW * CH)) * (NW * CH)
    dst3d = _pad_edges(edge_index[1], epd).reshape(NW, epd // (NW * CH), CH)
    parts = _sc_degree_kernel(epd // (NW * CH))(dst3d)
    deg = 1.0 + parts[0, :N] + parts[1, :N]
    dinv = lax.rsqrt(deg)

    loop = jnp.arange(n, dtype=edge_index.dtype)
    etot = E + n
    ep = ((etot + NW * CH - 1) // (NW * CH)) * (NW * CH)
    npad_e = ep - etot
    src_pad = jnp.asarray((np.arange(npad_e) * 61) % N, jnp.int32)
    dst_pad = jnp.asarray(N + (np.arange(npad_e) % (NPAD - N)), jnp.int32)
    epw_chunks = ep // (NW * CH)
    src3d = jnp.concatenate([edge_index[0], loop, src_pad]).reshape(NW, epw_chunks, CH)
    dst3d = jnp.concatenate([edge_index[1], loop, dst_pad]).reshape(NW, epw_chunks, CH)
    layer_scatter = _sc_layer_kernel(epw_chunks)

    xp = jnp.pad(x, ((0, 0), (0, 5)))
    w1p = jnp.pad(W1, ((0, 5), (0, 0)))
    s_e = jnp.asarray(_S_EXPAND)
    t_e = jnp.asarray(_T_EXPAND)
    d0 = parts[0, :N, None]
    d1 = parts[1, :N, None]
    t0, eps_, epss_ = _tc_embed_a(xp, w1p, b1[None, :])
    h, dinv2, bases, wt = _tc_embed_b(
        t0, eps_, epss_, g1[None, :], be1[None, :], d0, d1,
        convWb[0], convWc[0], convbc[0][None, :])
    for l in range(LAYERS):
        ps = layer_scatter(src3d, dst3d, bases)
        o, ops_, opss_ = _tc_tail_a(
            dinv2, wt, ps[0, :N], ps[1, :N], convbias[l][None, :], s_e, t_e)
        ln = min(l + 1, LAYERS - 1)
        h, bases, wt = _tc_tail_b(
            h, o, ops_, opss_, convg[l][None, :], convbe[l][None, :], dinv2,
            convWb[ln], convWc[ln], convbc[ln][None, :])
    ssum, cnt = _tc_pool(h, batch[:, None])
    return _tc_head(
        ssum, cnt, descriptors, Wm1, gm1[None, :], bm1[None, :],
        Wm2, gm2[None, :], bm2[None, :], W2[:HID // 4], W2[HID // 4:],
        b2[None, :], g3[None, :], be3[None, :], Wout, bout[None, :])


# speedup vs baseline: 1.2194x; 1.2194x over previous
import functools

import jax
import jax.numpy as jnp
import numpy as np
from jax import lax
from jax.experimental import pallas as pl
from jax.experimental.pallas import tpu as pltpu
from jax.experimental.pallas import tpu_sc as plsc

N = 10000
NPAD = 10240
HID = 128
LAYERS = 4
HEADS = 8
BASES = 4
F = HID // HEADS
DESC = 200
NGRAPH = 128

NC = 2
NS = 16
NW = NC * NS
CH = 128


def _sc_degree_kernel(epw_chunks):
    mesh = plsc.VectorSubcoreMesh(core_axis_name="c", subcore_axis_name="s")
    rows_per_s = NPAD // NS

    @functools.partial(
        pl.kernel,
        mesh=mesh,
        out_type=jax.ShapeDtypeStruct((NC, NPAD), jnp.float32),
        scratch_types=[
            pltpu.VMEM((epw_chunks, CH), jnp.int32),
            pltpu.VMEM((CH,), jnp.float32),
            pltpu.VMEM((rows_per_s,), jnp.float32),
            pltpu.VMEM_SHARED((NPAD,), jnp.float32),
        ],
    )
    def k(dst_hbm, out_hbm, idx_v, ones_v, zeros_v, cnt_sh):
        c = lax.axis_index("c")
        s = lax.axis_index("s")
        wid = s * NC + c
        for i in range(CH // 16):
            ones_v[pl.ds(i * 16, 16)] = jnp.ones((16,), jnp.float32)

        def zbody(i, carry):
            zeros_v[pl.ds(i * 16, 16)] = jnp.zeros((16,), jnp.float32)
            return carry

        lax.fori_loop(0, rows_per_s // 16, zbody, 0)
        pltpu.sync_copy(zeros_v, cnt_sh.at[pl.ds(s * rows_per_s, rows_per_s)])
        pltpu.sync_copy(dst_hbm.at[wid], idx_v)
        plsc.subcore_barrier()

        def body(j, carry):
            pltpu.sync_copy(ones_v, cnt_sh.at[idx_v.at[j]], add=True)
            return carry

        lax.fori_loop(0, epw_chunks, body, 0)
        plsc.subcore_barrier()
        pltpu.sync_copy(
            cnt_sh.at[pl.ds(s * rows_per_s, rows_per_s)],
            out_hbm.at[c, pl.ds(s * rows_per_s, rows_per_s)],
        )

    return k


def _sc_layer_kernel(epw_chunks):
    mesh = plsc.VectorSubcoreMesh(core_axis_name="c", subcore_axis_name="s")
    rows_per_s = NPAD // NS

    @functools.partial(
        pl.kernel,
        mesh=mesh,
        out_type=jax.ShapeDtypeStruct((NC, NPAD, BASES * F), jnp.float32),
        compiler_params=pltpu.CompilerParams(use_tc_tiling_on_sc=False),
        scratch_types=[
            pltpu.VMEM((epw_chunks, CH), jnp.int32),
            pltpu.VMEM((epw_chunks, CH), jnp.int32),
            pltpu.VMEM((CH, BASES * F), jnp.float32),
            pltpu.VMEM((CH, BASES * F), jnp.float32),
            pltpu.VMEM((CH, BASES * F), jnp.float32),
            pltpu.VMEM_SHARED((NPAD, BASES * F), jnp.float32),
            pltpu.SemaphoreType.DMA,
            pltpu.SemaphoreType.DMA,
        ],
    )
    def k(src_hbm, dst_hbm, bases_hbm, out_hbm, sidx_v, didx_v, zeros_v,
          rows0_v, rows1_v, agg_sh, sem0, sem1):
        c = lax.axis_index("c")
        s = lax.axis_index("s")
        wid = s * NC + c

        def zb(i, carry):
            def zb2(j, carry2):
                zeros_v[i, pl.ds(j * 16, 16)] = jnp.zeros((16,), jnp.float32)
                return carry2
            return lax.fori_loop(0, (BASES * F) // 16, zb2, carry)

        lax.fori_loop(0, CH, zb, 0)
        def zcopy(i, carry):
            pltpu.sync_copy(zeros_v, agg_sh.at[pl.ds(s * rows_per_s + i * CH, CH)])
            return carry

        lax.fori_loop(0, rows_per_s // CH, zcopy, 0)
        pltpu.sync_copy(src_hbm.at[wid], sidx_v)
        pltpu.sync_copy(dst_hbm.at[wid], didx_v)
        plsc.subcore_barrier()

        pltpu.async_copy(bases_hbm.at[sidx_v.at[0]], rows0_v, sem0)

        def body(j2, carry):
            j = j2 * 2
            pltpu.async_copy(bases_hbm.at[sidx_v.at[j + 1]], rows1_v, sem1)
            pltpu.make_async_copy(bases_hbm.at[sidx_v.at[j]], rows0_v,
                                  sem0).wait()
            pltpu.sync_copy(rows0_v, agg_sh.at[didx_v.at[j]], add=True)

            @pl.when(j + 2 < epw_chunks)
            def _():
                pltpu.async_copy(bases_hbm.at[sidx_v.at[j + 2]], rows0_v,
                                 sem0)

            pltpu.make_async_copy(bases_hbm.at[sidx_v.at[j + 1]], rows1_v,
                                  sem1).wait()
            pltpu.sync_copy(rows1_v, agg_sh.at[didx_v.at[j + 1]], add=True)
            return carry

        lax.fori_loop(0, epw_chunks // 2, body, 0)
        plsc.subcore_barrier()
        pltpu.sync_copy(
            agg_sh.at[pl.ds(s * rows_per_s, rows_per_s)],
            out_hbm.at[c, pl.ds(s * rows_per_s, rows_per_s)],
        )

    return k


def _pad_edges(idx, total):
    pad = total - idx.shape[0]
    pad_rows = jnp.asarray(N + (np.arange(pad) % (NPAD - N)), jnp.int32)
    return jnp.concatenate([idx, pad_rows])


def _bn(x, g, b, eps=1e-5):
    mu = jnp.mean(x, axis=0, keepdims=True)
    var = jnp.mean((x - mu) * (x - mu), axis=0, keepdims=True)
    return (x - mu) * lax.rsqrt(var + eps) * g + b


def _dot(a, b):
    return jax.lax.dot_general(
        a, b, (((1,), (0,)), ((), ())),
        preferred_element_type=jnp.float32)


def _dotT(a, b):
    return jax.lax.dot_general(
        a, b, (((0,), (0,)), ((), ())),
        preferred_element_type=jnp.float32)


_S_EXPAND = np.zeros((BASES, BASES * HEADS, HID), np.float32)
_T_EXPAND = np.zeros((BASES, BASES * F, HID), np.float32)
for _b in range(BASES):
    for _h in range(HEADS):
        for _f in range(F):
            _S_EXPAND[_b, _h * BASES + _b, _h * F + _f] = 1.0
            _T_EXPAND[_b, _b * F + _f, _h * F + _f] = 1.0


RB = 2000
NBLK = N // RB


def _blk(cols):
    return pl.BlockSpec((RB, cols), lambda i: (i, 0))


def _full(shape):
    nd = len(shape)
    return pl.BlockSpec(shape, lambda i: (0,) * nd)


def _tc_embed_a(xp, w1p, b1):
    def body(x_r, w1_r, b1_r, t_r, ps_r, pss_r):
        t = _dot(x_r[...], w1_r[...]) + b1_r[...]
        t_r[...] = t
        mb = jnp.sum(t, axis=0, keepdims=True) / RB
        d = t - mb
        ps_r[...] = mb.reshape(1, 1, HID)
        pss_r[...] = jnp.sum(d * d, axis=0, keepdims=True).reshape(1, 1, HID)

    return pl.pallas_call(
        body,
        grid=(NBLK,),
        in_specs=[_blk(32), _full((32, HID)), _full((1, HID))],
        out_specs=[_blk(HID), pl.BlockSpec((1, 1, HID), lambda i: (i, 0, 0)),
                   pl.BlockSpec((1, 1, HID), lambda i: (i, 0, 0))],
        out_shape=[
            jax.ShapeDtypeStruct((N, HID), jnp.float32),
            jax.ShapeDtypeStruct((NBLK, 1, HID), jnp.float32),
            jax.ShapeDtypeStruct((NBLK, 1, HID), jnp.float32),
        ],
    )(xp, w1p, b1)


def _finalize_stats(ps, pss, eps=1e-5):
    mu = jnp.sum(ps[...], axis=0) / NBLK
    dm = ps[...] - mu
    var = (jnp.sum(pss[...], axis=0) + RB * jnp.sum(dm * dm, axis=0)) / N
    return mu, lax.rsqrt(var + eps)


def _tc_embed_b(t, ps, pss, g1, be1, d0, d1, wb, wc, bc):
    def body(t_r, ps_r, pss_r, g_r, be_r, d0_r, d1_r, wb_r, wc_r, bc_r,
             h_r, dinv_r, bases_r, wt_r):
        mu, rstd = _finalize_stats(ps_r, pss_r)
        h = jax.nn.relu((t_r[...] - mu) * rstd * g_r[...] + be_r[...])
        h_r[...] = h
        dinv = lax.rsqrt(1.0 + d0_r[...] + d1_r[...])
        dinv_r[...] = dinv
        bases_r[...] = dinv * _dot(h, wb_r[...])
        wt_r[...] = _dot(h, wc_r[...]) + bc_r[...]

    return pl.pallas_call(
        body,
        grid=(NBLK,),
        in_specs=[_blk(HID), _full((NBLK, 1, HID)), _full((NBLK, 1, HID)),
                  _full((1, HID)), _full((1, HID)), _blk(1), _blk(1),
                  _full((HID, BASES * F)), _full((HID, HEADS * BASES)),
                  _full((1, HEADS * BASES))],
        out_specs=[_blk(HID), _blk(1), _blk(BASES * F), _blk(HEADS * BASES)],
        out_shape=[
            jax.ShapeDtypeStruct((N, HID), jnp.float32),
            jax.ShapeDtypeStruct((N, 1), jnp.float32),
            jax.ShapeDtypeStruct((N, BASES * F), jnp.float32),
            jax.ShapeDtypeStruct((N, HEADS * BASES), jnp.float32),
        ],
    )(t, ps, pss, g1, be1, d0, d1, wb, wc, bc)


def _tc_tail_a(dinv, wt, p0, p1, bias, s_e, t_e):
    def body(dinv_r, wt_r, p0_r, p1_r, bias_r, s_r, t_r, o_r, ps_r, pss_r):
        agg = dinv_r[...] * (p0_r[...] + p1_r[...])
        wt = wt_r[...]
        cols = []
        for h in range(HEADS):
            acc = None
            for b in range(BASES):
                c = h * BASES + b
                term = wt[:, c:c + 1] * agg[:, b * F:(b + 1) * F]
                acc = term if acc is None else acc + term
            cols.append(acc)
        o = jnp.concatenate(cols, axis=1) + bias_r[...]
        o_r[...] = o
        mb = jnp.sum(o, axis=0, keepdims=True) / RB
        d = o - mb
        ps_r[...] = mb.reshape(1, 1, HID)
        pss_r[...] = jnp.sum(d * d, axis=0, keepdims=True).reshape(1, 1, HID)

    return pl.pallas_call(
        body,
        grid=(NBLK,),
        in_specs=[_blk(1), _blk(HEADS * BASES), _blk(BASES * F),
                  _blk(BASES * F), _full((1, HID)),
                  _full((BASES, HEADS * BASES, HID)),
                  _full((BASES, BASES * F, HID))],
        out_specs=[_blk(HID), pl.BlockSpec((1, 1, HID), lambda i: (i, 0, 0)),
                   pl.BlockSpec((1, 1, HID), lambda i: (i, 0, 0))],
        out_shape=[
            jax.ShapeDtypeStruct((N, HID), jnp.float32),
            jax.ShapeDtypeStruct((NBLK, 1, HID), jnp.float32),
            jax.ShapeDtypeStruct((NBLK, 1, HID), jnp.float32),
        ],
    )(dinv, wt, p0, p1, bias, s_e, t_e)


def _tc_tail_b(h, o, ps, pss, g, be, dinv, wb, wc, bc):
    def body(h_r, o_r, ps_r, pss_r, g_r, be_r, dinv_r, wb_r, wc_r, bc_r,
             hn_r, bases_r, wt_r):
        mu, rstd = _finalize_stats(ps_r, pss_r)
        hn = h_r[...] + jax.nn.relu((o_r[...] - mu) * rstd * g_r[...]
                                    + be_r[...])
        hn_r[...] = hn
        bases_r[...] = dinv_r[...] * _dot(hn, wb_r[...])
        wt_r[...] = _dot(hn, wc_r[...]) + bc_r[...]

    return pl.pallas_call(
        body,
        grid=(NBLK,),
        in_specs=[_blk(HID), _blk(HID), _full((NBLK, 1, HID)),
                  _full((NBLK, 1, HID)), _full((1, HID)), _full((1, HID)),
                  _blk(1), _full((HID, BASES * F)),
                  _full((HID, HEADS * BASES)), _full((1, HEADS * BASES))],
        out_specs=[_blk(HID), _blk(BASES * F), _blk(HEADS * BASES)],
        out_shape=[
            jax.ShapeDtypeStruct((N, HID), jnp.float32),
            jax.ShapeDtypeStruct((N, BASES * F), jnp.float32),
            jax.ShapeDtypeStruct((N, HEADS * BASES), jnp.float32),
        ],
    )(h, o, ps, pss, g, be, dinv, wb, wc, bc)


def _tc_pool(hn, batch_col):
    def body(h_r, batch_r, ssum_r, cnt_r):
        i = pl.program_id(0)
        gids = jax.lax.broadcasted_iota(jnp.int32, (1, NGRAPH), 1)
        onehot = (batch_r[...] == gids).astype(jnp.float32)

        @pl.when(i == 0)
        def _():
            ssum_r[...] = jnp.zeros((NGRAPH, HID), jnp.float32)
            cnt_r[...] = jnp.zeros((NGRAPH, 1), jnp.float32)

        ssum_r[...] += _dotT(onehot, h_r[...])
        cnt_r[...] += _dotT(onehot, jnp.ones((RB, 1), jnp.float32))

    return pl.pallas_call(
        body,
        grid=(NBLK,),
        in_specs=[_blk(HID), _blk(1)],
        out_specs=[_full((NGRAPH, HID)), _full((NGRAPH, 1))],
        out_shape=[
            jax.ShapeDtypeStruct((NGRAPH, HID), jnp.float32),
            jax.ShapeDtypeStruct((NGRAPH, 1), jnp.float32),
        ],
    )(hn, batch_col)


def _tc_head(ssum, cnt, descriptors, Wm1, gm1, bm1, Wm2, gm2, bm2, W2m,
             W2d, b2, g3, be3, Wout, bout):
    def body(ssum_r, cnt_r, desc_r, wm1_r, gm1_r, bm1_r, wm2_r, gm2_r,
             bm2_r, w2m_r, w2d_r, b2_r, g3_r, be3_r, wout_r, bout_r, out_r):
        pooled = ssum_r[...] * (1.0 / jnp.maximum(cnt_r[...], 1.0))
        m = jax.nn.relu(_bn(_dot(pooled, wm1_r[...]), gm1_r[...], bm1_r[...]))
        m = jax.nn.relu(_bn(_dot(m, wm2_r[...]), gm2_r[...], bm2_r[...]))
        z = jax.nn.relu(_dot(m, w2m_r[...]) + _dot(desc_r[...], w2d_r[...])
                        + b2_r[...])
        z = _bn(z, g3_r[...], be3_r[...])
        out_r[...] = _dot(z, wout_r[...]) + bout_r[...]

    return pl.pallas_call(
        body,
        out_shape=jax.ShapeDtypeStruct((NGRAPH, 1), jnp.float32),
    )(ssum, cnt, descriptors, Wm1, gm1, bm1, Wm2, gm2, bm2, W2m, W2d,
      b2, g3, be3, Wout, bout)


def kernel(x, edge_index, batch, descriptors, W1, b1, g1, be1, convWb, convWc,
           convbc, convbias, convg, convbe, Wm1, gm1, bm1, Wm2, gm2, bm2, W2,
           b2, g3, be3, Wout, bout):
    n = N
    E = edge_index.shape[1]
    epd = ((E + NW * CH - 1) // (NW * CH)) * (NW * CH)
    dst3d = _pad_edges(edge_index[1], epd).reshape(NW, epd // (NW * CH), CH)
    parts = _sc_degree_kernel(epd // (NW * CH))(dst3d)
    deg = 1.0 + parts[0, :N] + parts[1, :N]
    dinv = lax.rsqrt(deg)

    loop = jnp.arange(n, dtype=edge_index.dtype)
    etot = E + n
    ep = ((etot + 2 * NW * CH - 1) // (2 * NW * CH)) * (2 * NW * CH)
    npad_e = ep - etot
    src_pad = jnp.asarray((np.arange(npad_e) * 61) % N, jnp.int32)
    dst_pad = jnp.asarray(N + (np.arange(npad_e) % (NPAD - N)), jnp.int32)
    epw_chunks = ep // (NW * CH)
    src3d = jnp.concatenate([edge_index[0], loop, src_pad]).reshape(NW, epw_chunks, CH)
    dst3d = jnp.concatenate([edge_index[1], loop, dst_pad]).reshape(NW, epw_chunks, CH)
    layer_scatter = _sc_layer_kernel(epw_chunks)

    xp = jnp.pad(x, ((0, 0), (0, 5)))
    w1p = jnp.pad(W1, ((0, 5), (0, 0)))
    s_e = jnp.asarray(_S_EXPAND)
    t_e = jnp.asarray(_T_EXPAND)
    d0 = parts[0, :N, None]
    d1 = parts[1, :N, None]
    t0, eps_, epss_ = _tc_embed_a(xp, w1p, b1[None, :])
    h, dinv2, bases, wt = _tc_embed_b(
        t0, eps_, epss_, g1[None, :], be1[None, :], d0, d1,
        convWb[0], convWc[0], convbc[0][None, :])
    for l in range(LAYERS):
        ps = layer_scatter(src3d, dst3d, bases)
        o, ops_, opss_ = _tc_tail_a(
            dinv2, wt, ps[0, :N], ps[1, :N], convbias[l][None, :], s_e, t_e)
        ln = min(l + 1, LAYERS - 1)
        h, bases, wt = _tc_tail_b(
            h, o, ops_, opss_, convg[l][None, :], convbe[l][None, :], dinv2,
            convWb[ln], convWc[ln], convbc[ln][None, :])
    ssum, cnt = _tc_pool(h, batch[:, None])
    return _tc_head(
        ssum, cnt, descriptors, Wm1, gm1[None, :], bm1[None, :],
        Wm2, gm2[None, :], bm2[None, :], W2[:HID // 4], W2[HID // 4:],
        b2[None, :], g3[None, :], be3[None, :], Wout, bout[None, :])


# speedup vs baseline: 1.9097x; 1.5661x over previous
import functools

import jax
import jax.numpy as jnp
import numpy as np
from jax import lax
from jax.experimental import pallas as pl
from jax.experimental.pallas import tpu as pltpu
from jax.experimental.pallas import tpu_sc as plsc

N = 10000
NPAD = 10240
HID = 128
LAYERS = 4
HEADS = 8
BASES = 4
F = HID // HEADS
DESC = 200
NGRAPH = 128

NC = 2
NS = 16
NW = NC * NS
CH = 128


def _sc_degree_kernel(epw_chunks):
    mesh = plsc.VectorSubcoreMesh(core_axis_name="c", subcore_axis_name="s")
    rows_per_s = NPAD // NS

    @functools.partial(
        pl.kernel,
        mesh=mesh,
        out_type=jax.ShapeDtypeStruct((NC, NPAD), jnp.float32),
        scratch_types=[
            pltpu.VMEM((epw_chunks, CH), jnp.int32),
            pltpu.VMEM((CH,), jnp.float32),
            pltpu.VMEM((rows_per_s,), jnp.float32),
            pltpu.VMEM_SHARED((NPAD,), jnp.float32),
        ],
    )
    def k(dst_hbm, out_hbm, idx_v, ones_v, zeros_v, cnt_sh):
        c = lax.axis_index("c")
        s = lax.axis_index("s")
        wid = s * NC + c
        for i in range(CH // 16):
            ones_v[pl.ds(i * 16, 16)] = jnp.ones((16,), jnp.float32)

        def zbody(i, carry):
            zeros_v[pl.ds(i * 16, 16)] = jnp.zeros((16,), jnp.float32)
            return carry

        lax.fori_loop(0, rows_per_s // 16, zbody, 0)
        pltpu.sync_copy(zeros_v, cnt_sh.at[pl.ds(s * rows_per_s, rows_per_s)])
        pltpu.sync_copy(dst_hbm.at[wid], idx_v)
        plsc.subcore_barrier()

        def body(j, carry):
            pltpu.sync_copy(ones_v, cnt_sh.at[idx_v.at[j]], add=True)
            return carry

        lax.fori_loop(0, epw_chunks, body, 0)
        plsc.subcore_barrier()
        pltpu.sync_copy(
            cnt_sh.at[pl.ds(s * rows_per_s, rows_per_s)],
            out_hbm.at[c, pl.ds(s * rows_per_s, rows_per_s)],
        )

    return k


def _sc_layer_kernel(epw_chunks):
    mesh = plsc.VectorSubcoreMesh(core_axis_name="c", subcore_axis_name="s")
    rows_per_s = NPAD // NS

    @functools.partial(
        pl.kernel,
        mesh=mesh,
        out_type=jax.ShapeDtypeStruct((NC, NPAD, BASES * F), jnp.float32),
        compiler_params=pltpu.CompilerParams(use_tc_tiling_on_sc=False),
        scratch_types=[
            pltpu.VMEM((epw_chunks, CH), jnp.int32),
            pltpu.VMEM((epw_chunks, CH), jnp.int32),
            pltpu.VMEM((CH, BASES * F), jnp.float32),
            pltpu.VMEM((CH, BASES * F), jnp.float32),
            pltpu.VMEM((CH, BASES * F), jnp.float32),
            pltpu.VMEM_SHARED((NPAD, BASES * F), jnp.float32),
            pltpu.SemaphoreType.DMA,
            pltpu.SemaphoreType.DMA,
        ],
    )
    def k(src_hbm, dst_hbm, bases_hbm, out_hbm, sidx_v, didx_v, zeros_v,
          rows0_v, rows1_v, agg_sh, sem0, sem1):
        c = lax.axis_index("c")
        s = lax.axis_index("s")
        wid = s * NC + c

        def zb(i, carry):
            def zb2(j, carry2):
                zeros_v[i, pl.ds(j * 16, 16)] = jnp.zeros((16,), jnp.float32)
                return carry2
            return lax.fori_loop(0, (BASES * F) // 16, zb2, carry)

        lax.fori_loop(0, CH, zb, 0)
        def zcopy(i, carry):
            pltpu.sync_copy(zeros_v, agg_sh.at[pl.ds(s * rows_per_s + i * CH, CH)])
            return carry

        lax.fori_loop(0, rows_per_s // CH, zcopy, 0)
        pltpu.sync_copy(src_hbm.at[wid], sidx_v)
        pltpu.sync_copy(dst_hbm.at[wid], didx_v)
        plsc.subcore_barrier()

        pltpu.async_copy(bases_hbm.at[sidx_v.at[0]], rows0_v, sem0)

        def body(j2, carry):
            j = j2 * 2
            pltpu.async_copy(bases_hbm.at[sidx_v.at[j + 1]], rows1_v, sem1)
            pltpu.make_async_copy(bases_hbm.at[sidx_v.at[j]], rows0_v,
                                  sem0).wait()
            pltpu.sync_copy(rows0_v, agg_sh.at[didx_v.at[j]], add=True)

            @pl.when(j + 2 < epw_chunks)
            def _():
                pltpu.async_copy(bases_hbm.at[sidx_v.at[j + 2]], rows0_v,
                                 sem0)

            pltpu.make_async_copy(bases_hbm.at[sidx_v.at[j + 1]], rows1_v,
                                  sem1).wait()
            pltpu.sync_copy(rows1_v, agg_sh.at[didx_v.at[j + 1]], add=True)
            return carry

        lax.fori_loop(0, epw_chunks // 2, body, 0)
        plsc.subcore_barrier()
        pltpu.sync_copy(
            agg_sh.at[pl.ds(s * rows_per_s, rows_per_s)],
            out_hbm.at[c, pl.ds(s * rows_per_s, rows_per_s)],
        )

    return k


def _pad_edges(idx, total):
    pad = total - idx.shape[0]
    pad_rows = jnp.asarray(N + (np.arange(pad) % (NPAD - N)), jnp.int32)
    return jnp.concatenate([idx, pad_rows])


def _bn(x, g, b, eps=1e-5):
    mu = jnp.mean(x, axis=0, keepdims=True)
    var = jnp.mean((x - mu) * (x - mu), axis=0, keepdims=True)
    return (x - mu) * lax.rsqrt(var + eps) * g + b


def _dot(a, b):
    return jax.lax.dot_general(
        a, b, (((1,), (0,)), ((), ())),
        preferred_element_type=jnp.float32)


def _dotT(a, b):
    return jax.lax.dot_general(
        a, b, (((0,), (0,)), ((), ())),
        preferred_element_type=jnp.float32)


_S_EXPAND = np.zeros((BASES, BASES * HEADS, HID), np.float32)
_T_EXPAND = np.zeros((BASES, BASES * F, HID), np.float32)
for _b in range(BASES):
    for _h in range(HEADS):
        for _f in range(F):
            _S_EXPAND[_b, _h * BASES + _b, _h * F + _f] = 1.0
            _T_EXPAND[_b, _b * F + _f, _h * F + _f] = 1.0


RB = 2000
NBLK = N // RB


def _blk(cols):
    return pl.BlockSpec((RB, cols), lambda i: (i, 0))


def _full(shape):
    nd = len(shape)
    return pl.BlockSpec(shape, lambda i: (0,) * nd)


def _tc_embed_a(xp, w1p, b1):
    def body(x_r, w1_r, b1_r, t_r, ps_r, pss_r):
        t = _dot(x_r[...], w1_r[...]) + b1_r[...]
        t_r[...] = t
        mb = jnp.sum(t, axis=0, keepdims=True) / RB
        d = t - mb
        ps_r[...] = mb.reshape(1, 1, HID)
        pss_r[...] = jnp.sum(d * d, axis=0, keepdims=True).reshape(1, 1, HID)

    return pl.pallas_call(
        body,
        grid=(NBLK,),
        in_specs=[_blk(32), _full((32, HID)), _full((1, HID))],
        out_specs=[_blk(HID), pl.BlockSpec((1, 1, HID), lambda i: (i, 0, 0)),
                   pl.BlockSpec((1, 1, HID), lambda i: (i, 0, 0))],
        out_shape=[
            jax.ShapeDtypeStruct((N, HID), jnp.float32),
            jax.ShapeDtypeStruct((NBLK, 1, HID), jnp.float32),
            jax.ShapeDtypeStruct((NBLK, 1, HID), jnp.float32),
        ],
    )(xp, w1p, b1)


def _finalize_stats(ps, pss, eps=1e-5):
    mu = jnp.sum(ps[...], axis=0) / NBLK
    dm = ps[...] - mu
    var = (jnp.sum(pss[...], axis=0) + RB * jnp.sum(dm * dm, axis=0)) / N
    return mu, lax.rsqrt(var + eps)


def _tc_embed_b(t, ps, pss, g1, be1, d0, d1, wb, wc, bc):
    def body(t_r, ps_r, pss_r, g_r, be_r, d0_r, d1_r, wb_r, wc_r, bc_r,
             h_r, dinv_r, bases_r, wt_r):
        mu, rstd = _finalize_stats(ps_r, pss_r)
        h = jax.nn.relu((t_r[...] - mu) * rstd * g_r[...] + be_r[...])
        h_r[...] = h
        dinv = lax.rsqrt(1.0 + d0_r[...] + d1_r[...])
        dinv_r[...] = dinv
        bases_r[...] = dinv * _dot(h, wb_r[...])
        wt_r[...] = _dot(h, wc_r[...]) + bc_r[...]

    return pl.pallas_call(
        body,
        grid=(NBLK,),
        in_specs=[_blk(HID), _full((NBLK, 1, HID)), _full((NBLK, 1, HID)),
                  _full((1, HID)), _full((1, HID)), _blk(1), _blk(1),
                  _full((HID, BASES * F)), _full((HID, HEADS * BASES)),
                  _full((1, HEADS * BASES))],
        out_specs=[_blk(HID), _blk(1), _blk(BASES * F), _blk(HEADS * BASES)],
        out_shape=[
            jax.ShapeDtypeStruct((N, HID), jnp.float32),
            jax.ShapeDtypeStruct((N, 1), jnp.float32),
            jax.ShapeDtypeStruct((N, BASES * F), jnp.float32),
            jax.ShapeDtypeStruct((N, HEADS * BASES), jnp.float32),
        ],
    )(t, ps, pss, g1, be1, d0, d1, wb, wc, bc)


def _tc_tail_a(dinv, wt, p0, p1, bias, s_e, t_e):
    def body(dinv_r, wt_r, p0_r, p1_r, bias_r, s_r, t_r, o_r, ps_r, pss_r):
        agg = dinv_r[...] * (p0_r[...] + p1_r[...])
        wt = wt_r[...]
        o = bias_r[...] + _dot(wt, s_r[0]) * _dot(agg, t_r[0])
        for b in range(1, BASES):
            o = o + _dot(wt, s_r[b]) * _dot(agg, t_r[b])
        o_r[...] = o
        mb = jnp.sum(o, axis=0, keepdims=True) / RB
        d = o - mb
        ps_r[...] = mb.reshape(1, 1, HID)
        pss_r[...] = jnp.sum(d * d, axis=0, keepdims=True).reshape(1, 1, HID)

    return pl.pallas_call(
        body,
        grid=(NBLK,),
        in_specs=[_blk(1), _blk(HEADS * BASES), _blk(BASES * F),
                  _blk(BASES * F), _full((1, HID)),
                  _full((BASES, HEADS * BASES, HID)),
                  _full((BASES, BASES * F, HID))],
        out_specs=[_blk(HID), pl.BlockSpec((1, 1, HID), lambda i: (i, 0, 0)),
                   pl.BlockSpec((1, 1, HID), lambda i: (i, 0, 0))],
        out_shape=[
            jax.ShapeDtypeStruct((N, HID), jnp.float32),
            jax.ShapeDtypeStruct((NBLK, 1, HID), jnp.float32),
            jax.ShapeDtypeStruct((NBLK, 1, HID), jnp.float32),
        ],
    )(dinv, wt, p0, p1, bias, s_e, t_e)


def _tc_tail_b(h, o, ps, pss, g, be, dinv, wb, wc, bc):
    def body(h_r, o_r, ps_r, pss_r, g_r, be_r, dinv_r, wb_r, wc_r, bc_r,
             hn_r, bases_r, wt_r):
        mu, rstd = _finalize_stats(ps_r, pss_r)
        hn = h_r[...] + jax.nn.relu((o_r[...] - mu) * rstd * g_r[...]
                                    + be_r[...])
        hn_r[...] = hn
        bases_r[...] = dinv_r[...] * _dot(hn, wb_r[...])
        wt_r[...] = _dot(hn, wc_r[...]) + bc_r[...]

    return pl.pallas_call(
        body,
        grid=(NBLK,),
        in_specs=[_blk(HID), _blk(HID), _full((NBLK, 1, HID)),
                  _full((NBLK, 1, HID)), _full((1, HID)), _full((1, HID)),
                  _blk(1), _full((HID, BASES * F)),
                  _full((HID, HEADS * BASES)), _full((1, HEADS * BASES))],
        out_specs=[_blk(HID), _blk(BASES * F), _blk(HEADS * BASES)],
        out_shape=[
            jax.ShapeDtypeStruct((N, HID), jnp.float32),
            jax.ShapeDtypeStruct((N, BASES * F), jnp.float32),
            jax.ShapeDtypeStruct((N, HEADS * BASES), jnp.float32),
        ],
    )(h, o, ps, pss, g, be, dinv, wb, wc, bc)


def _tc_pool(hn, batch_col):
    def body(h_r, batch_r, ssum_r, cnt_r):
        i = pl.program_id(0)
        gids = jax.lax.broadcasted_iota(jnp.int32, (1, NGRAPH), 1)
        onehot = (batch_r[...] == gids).astype(jnp.float32)

        @pl.when(i == 0)
        def _():
            ssum_r[...] = jnp.zeros((NGRAPH, HID), jnp.float32)
            cnt_r[...] = jnp.zeros((NGRAPH, 1), jnp.float32)

        ssum_r[...] += _dotT(onehot, h_r[...])
        cnt_r[...] += _dotT(onehot, jnp.ones((RB, 1), jnp.float32))

    return pl.pallas_call(
        body,
        grid=(NBLK,),
        in_specs=[_blk(HID), _blk(1)],
        out_specs=[_full((NGRAPH, HID)), _full((NGRAPH, 1))],
        out_shape=[
            jax.ShapeDtypeStruct((NGRAPH, HID), jnp.float32),
            jax.ShapeDtypeStruct((NGRAPH, 1), jnp.float32),
        ],
    )(hn, batch_col)


def _tc_head(ssum, cnt, descriptors, Wm1, gm1, bm1, Wm2, gm2, bm2, W2m,
             W2d, b2, g3, be3, Wout, bout):
    def body(ssum_r, cnt_r, desc_r, wm1_r, gm1_r, bm1_r, wm2_r, gm2_r,
             bm2_r, w2m_r, w2d_r, b2_r, g3_r, be3_r, wout_r, bout_r, out_r):
        pooled = ssum_r[...] * (1.0 / jnp.maximum(cnt_r[...], 1.0))
        m = jax.nn.relu(_bn(_dot(pooled, wm1_r[...]), gm1_r[...], bm1_r[...]))
        m = jax.nn.relu(_bn(_dot(m, wm2_r[...]), gm2_r[...], bm2_r[...]))
        z = jax.nn.relu(_dot(m, w2m_r[...]) + _dot(desc_r[...], w2d_r[...])
                        + b2_r[...])
        z = _bn(z, g3_r[...], be3_r[...])
        out_r[...] = _dot(z, wout_r[...]) + bout_r[...]

    return pl.pallas_call(
        body,
        out_shape=jax.ShapeDtypeStruct((NGRAPH, 1), jnp.float32),
    )(ssum, cnt, descriptors, Wm1, gm1, bm1, Wm2, gm2, bm2, W2m, W2d,
      b2, g3, be3, Wout, bout)


def kernel(x, edge_index, batch, descriptors, W1, b1, g1, be1, convWb, convWc,
           convbc, convbias, convg, convbe, Wm1, gm1, bm1, Wm2, gm2, bm2, W2,
           b2, g3, be3, Wout, bout):
    n = N
    E = edge_index.shape[1]
    epd = ((E + NW * CH - 1) // (NW * CH)) * (NW * CH)
    dst3d = _pad_edges(edge_index[1], epd).reshape(NW, epd // (NW * CH), CH)
    parts = _sc_degree_kernel(epd // (NW * CH))(dst3d)
    deg = 1.0 + parts[0, :N] + parts[1, :N]
    dinv = lax.rsqrt(deg)

    loop = jnp.arange(n, dtype=edge_index.dtype)
    etot = E + n
    ep = ((etot + 2 * NW * CH - 1) // (2 * NW * CH)) * (2 * NW * CH)
    npad_e = ep - etot
    src_pad = jnp.asarray((np.arange(npad_e) * 61) % N, jnp.int32)
    dst_pad = jnp.asarray(N + (np.arange(npad_e) % (NPAD - N)), jnp.int32)
    epw_chunks = ep // (NW * CH)
    src3d = jnp.concatenate([edge_index[0], loop, src_pad]).reshape(NW, epw_chunks, CH)
    dst3d = jnp.concatenate([edge_index[1], loop, dst_pad]).reshape(NW, epw_chunks, CH)
    layer_scatter = _sc_layer_kernel(epw_chunks)

    xp = jnp.pad(x, ((0, 0), (0, 5)))
    w1p = jnp.pad(W1, ((0, 5), (0, 0)))
    s_e = jnp.asarray(_S_EXPAND)
    t_e = jnp.asarray(_T_EXPAND)
    d0 = parts[0, :N, None]
    d1 = parts[1, :N, None]
    t0, eps_, epss_ = _tc_embed_a(xp, w1p, b1[None, :])
    h, dinv2, bases, wt = _tc_embed_b(
        t0, eps_, epss_, g1[None, :], be1[None, :], d0, d1,
        convWb[0], convWc[0], convbc[0][None, :])
    for l in range(LAYERS):
        ps = layer_scatter(src3d, dst3d, bases)
        o, ops_, opss_ = _tc_tail_a(
            dinv2, wt, ps[0, :N], ps[1, :N], convbias[l][None, :], s_e, t_e)
        ln = min(l + 1, LAYERS - 1)
        h, bases, wt = _tc_tail_b(
            h, o, ops_, opss_, convg[l][None, :], convbe[l][None, :], dinv2,
            convWb[ln], convWc[ln], convbc[ln][None, :])
    ssum, cnt = _tc_pool(h, batch[:, None])
    return _tc_head(
        ssum, cnt, descriptors, Wm1, gm1[None, :], bm1[None, :],
        Wm2, gm2[None, :], bm2[None, :], W2[:HID // 4], W2[HID // 4:],
        b2[None, :], g3[None, :], be3[None, :], Wout, bout[None, :])


# speedup vs baseline: 2.1240x; 1.1122x over previous
import functools

import jax
import jax.numpy as jnp
import numpy as np
from jax import lax
from jax.experimental import pallas as pl
from jax.experimental.pallas import tpu as pltpu
from jax.experimental.pallas import tpu_sc as plsc

N = 10000
NPAD = 10240
HID = 128
LAYERS = 4
HEADS = 8
BASES = 4
F = HID // HEADS
DESC = 200
NGRAPH = 128

NC = 2
NS = 16
NW = NC * NS
CH = 128


def _sc_degree_kernel(epw_chunks):
    mesh = plsc.VectorSubcoreMesh(core_axis_name="c", subcore_axis_name="s")
    rows_per_s = NPAD // NS

    @functools.partial(
        pl.kernel,
        mesh=mesh,
        out_type=jax.ShapeDtypeStruct((NC, NPAD), jnp.float32),
        scratch_types=[
            pltpu.VMEM((epw_chunks, CH), jnp.int32),
            pltpu.VMEM((CH,), jnp.float32),
            pltpu.VMEM((rows_per_s,), jnp.float32),
            pltpu.VMEM_SHARED((NPAD,), jnp.float32),
        ],
    )
    def k(dst_hbm, out_hbm, idx_v, ones_v, zeros_v, cnt_sh):
        c = lax.axis_index("c")
        s = lax.axis_index("s")
        wid = s * NC + c
        for i in range(CH // 16):
            ones_v[pl.ds(i * 16, 16)] = jnp.ones((16,), jnp.float32)

        def zbody(i, carry):
            zeros_v[pl.ds(i * 16, 16)] = jnp.zeros((16,), jnp.float32)
            return carry

        lax.fori_loop(0, rows_per_s // 16, zbody, 0)
        pltpu.sync_copy(zeros_v, cnt_sh.at[pl.ds(s * rows_per_s, rows_per_s)])
        pltpu.sync_copy(dst_hbm.at[wid], idx_v)
        plsc.subcore_barrier()

        def body(j, carry):
            pltpu.sync_copy(ones_v, cnt_sh.at[idx_v.at[j]], add=True)
            return carry

        lax.fori_loop(0, epw_chunks, body, 0)
        plsc.subcore_barrier()
        pltpu.sync_copy(
            cnt_sh.at[pl.ds(s * rows_per_s, rows_per_s)],
            out_hbm.at[c, pl.ds(s * rows_per_s, rows_per_s)],
        )

    return k


def _sc_layer_kernel(epw_chunks):
    mesh = plsc.VectorSubcoreMesh(core_axis_name="c", subcore_axis_name="s")
    rows_per_s = NPAD // NS

    @functools.partial(
        pl.kernel,
        mesh=mesh,
        out_type=jax.ShapeDtypeStruct((NC, NPAD, BASES * F), jnp.float32),
        compiler_params=pltpu.CompilerParams(use_tc_tiling_on_sc=False),
        scratch_types=[
            pltpu.VMEM((epw_chunks, CH), jnp.int32),
            pltpu.VMEM((epw_chunks, CH), jnp.int32),
            pltpu.VMEM((CH, BASES * F), jnp.float32),
            pltpu.VMEM((CH, BASES * F), jnp.float32),
            pltpu.VMEM((CH, BASES * F), jnp.float32),
            pltpu.VMEM((CH, BASES * F), jnp.float32),
            pltpu.VMEM_SHARED((NPAD, BASES * F), jnp.float32),
            pltpu.SemaphoreType.DMA,
            pltpu.SemaphoreType.DMA,
            pltpu.SemaphoreType.DMA,
        ],
    )
    def k(src_hbm, dst_hbm, bases_hbm, out_hbm, sidx_v, didx_v, zeros_v,
          rows0_v, rows1_v, rows2_v, agg_sh, sem0, sem1, sem2):
        c = lax.axis_index("c")
        s = lax.axis_index("s")
        wid = s * NC + c

        def zb(i, carry):
            def zb2(j, carry2):
                zeros_v[i, pl.ds(j * 16, 16)] = jnp.zeros((16,), jnp.float32)
                return carry2
            return lax.fori_loop(0, (BASES * F) // 16, zb2, carry)

        lax.fori_loop(0, CH, zb, 0)
        def zcopy(i, carry):
            pltpu.sync_copy(zeros_v, agg_sh.at[pl.ds(s * rows_per_s + i * CH, CH)])
            return carry

        lax.fori_loop(0, rows_per_s // CH, zcopy, 0)
        pltpu.sync_copy(src_hbm.at[wid], sidx_v)
        pltpu.sync_copy(dst_hbm.at[wid], didx_v)
        plsc.subcore_barrier()

        bufs = ((rows0_v, sem0), (rows1_v, sem1), (rows2_v, sem2))
        pltpu.async_copy(bases_hbm.at[sidx_v.at[0]], rows0_v, sem0)
        pltpu.async_copy(bases_hbm.at[sidx_v.at[1]], rows1_v, sem1)

        def body(j3, carry):
            j = j3 * 3
            for k in range(3):
                buf, sem = bufs[k]
                nbuf, nsem = bufs[(k + 2) % 3]
                if k == 0:
                    pltpu.async_copy(bases_hbm.at[sidx_v.at[j + 2]], nbuf,
                                     nsem)
                else:
                    @pl.when(j + k + 2 < epw_chunks)
                    def _():
                        pltpu.async_copy(bases_hbm.at[sidx_v.at[j + k + 2]],
                                         nbuf, nsem)

                pltpu.make_async_copy(bases_hbm.at[sidx_v.at[j + k]], buf,
                                      sem).wait()
                pltpu.sync_copy(buf, agg_sh.at[didx_v.at[j + k]], add=True)
            return carry

        lax.fori_loop(0, epw_chunks // 3, body, 0)
        plsc.subcore_barrier()
        pltpu.sync_copy(
            agg_sh.at[pl.ds(s * rows_per_s, rows_per_s)],
            out_hbm.at[c, pl.ds(s * rows_per_s, rows_per_s)],
        )

    return k


def _pad_edges(idx, total):
    pad = total - idx.shape[0]
    pad_rows = jnp.asarray(N + (np.arange(pad) % (NPAD - N)), jnp.int32)
    return jnp.concatenate([idx, pad_rows])


def _bn(x, g, b, eps=1e-5):
    mu = jnp.mean(x, axis=0, keepdims=True)
    var = jnp.mean((x - mu) * (x - mu), axis=0, keepdims=True)
    return (x - mu) * lax.rsqrt(var + eps) * g + b


def _dot(a, b):
    return jax.lax.dot_general(
        a, b, (((1,), (0,)), ((), ())),
        preferred_element_type=jnp.float32)


def _dotT(a, b):
    return jax.lax.dot_general(
        a, b, (((0,), (0,)), ((), ())),
        preferred_element_type=jnp.float32)


_S_EXPAND = np.zeros((BASES, BASES * HEADS, HID), np.float32)
_T_EXPAND = np.zeros((BASES, BASES * F, HID), np.float32)
for _b in range(BASES):
    for _h in range(HEADS):
        for _f in range(F):
            _S_EXPAND[_b, _h * BASES + _b, _h * F + _f] = 1.0
            _T_EXPAND[_b, _b * F + _f, _h * F + _f] = 1.0


RB = 2000
NBLK = N // RB


def _blk(cols):
    return pl.BlockSpec((RB, cols), lambda i: (i, 0))


def _full(shape):
    nd = len(shape)
    return pl.BlockSpec(shape, lambda i: (0,) * nd)


def _tc_embed_a(xp, w1p, b1):
    def body(x_r, w1_r, b1_r, t_r, ps_r, pss_r):
        t = _dot(x_r[...], w1_r[...]) + b1_r[...]
        t_r[...] = t
        mb = jnp.sum(t, axis=0, keepdims=True) / RB
        d = t - mb
        ps_r[...] = mb.reshape(1, 1, HID)
        pss_r[...] = jnp.sum(d * d, axis=0, keepdims=True).reshape(1, 1, HID)

    return pl.pallas_call(
        body,
        grid=(NBLK,),
        in_specs=[_blk(32), _full((32, HID)), _full((1, HID))],
        out_specs=[_blk(HID), pl.BlockSpec((1, 1, HID), lambda i: (i, 0, 0)),
                   pl.BlockSpec((1, 1, HID), lambda i: (i, 0, 0))],
        out_shape=[
            jax.ShapeDtypeStruct((N, HID), jnp.float32),
            jax.ShapeDtypeStruct((NBLK, 1, HID), jnp.float32),
            jax.ShapeDtypeStruct((NBLK, 1, HID), jnp.float32),
        ],
    )(xp, w1p, b1)


def _finalize_stats(ps, pss, eps=1e-5):
    mu = jnp.sum(ps[...], axis=0) / NBLK
    dm = ps[...] - mu
    var = (jnp.sum(pss[...], axis=0) + RB * jnp.sum(dm * dm, axis=0)) / N
    return mu, lax.rsqrt(var + eps)


def _tc_embed_b(t, ps, pss, g1, be1, d0, d1, wb, wc, bc):
    def body(t_r, ps_r, pss_r, g_r, be_r, d0_r, d1_r, wb_r, wc_r, bc_r,
             h_r, dinv_r, bases_r, wt_r):
        mu, rstd = _finalize_stats(ps_r, pss_r)
        h = jax.nn.relu((t_r[...] - mu) * rstd * g_r[...] + be_r[...])
        h_r[...] = h
        dinv = lax.rsqrt(1.0 + d0_r[...] + d1_r[...])
        dinv_r[...] = dinv
        bases_r[...] = dinv * _dot(h, wb_r[...])
        wt_r[...] = _dot(h, wc_r[...]) + bc_r[...]

    return pl.pallas_call(
        body,
        grid=(NBLK,),
        in_specs=[_blk(HID), _full((NBLK, 1, HID)), _full((NBLK, 1, HID)),
                  _full((1, HID)), _full((1, HID)), _blk(1), _blk(1),
                  _full((HID, BASES * F)), _full((HID, HEADS * BASES)),
                  _full((1, HEADS * BASES))],
        out_specs=[_blk(HID), _blk(1), _blk(BASES * F), _blk(HEADS * BASES)],
        out_shape=[
            jax.ShapeDtypeStruct((N, HID), jnp.float32),
            jax.ShapeDtypeStruct((N, 1), jnp.float32),
            jax.ShapeDtypeStruct((N, BASES * F), jnp.float32),
            jax.ShapeDtypeStruct((N, HEADS * BASES), jnp.float32),
        ],
    )(t, ps, pss, g1, be1, d0, d1, wb, wc, bc)


def _tc_tail_a(dinv, wt, p0, p1, bias, s_e, t_e):
    def body(dinv_r, wt_r, p0_r, p1_r, bias_r, s_r, t_r, o_r, ps_r, pss_r):
        agg = dinv_r[...] * (p0_r[...] + p1_r[...])
        wt = wt_r[...]
        o = bias_r[...] + _dot(wt, s_r[0]) * _dot(agg, t_r[0])
        for b in range(1, BASES):
            o = o + _dot(wt, s_r[b]) * _dot(agg, t_r[b])
        o_r[...] = o
        mb = jnp.sum(o, axis=0, keepdims=True) / RB
        d = o - mb
        ps_r[...] = mb.reshape(1, 1, HID)
        pss_r[...] = jnp.sum(d * d, axis=0, keepdims=True).reshape(1, 1, HID)

    return pl.pallas_call(
        body,
        grid=(NBLK,),
        in_specs=[_blk(1), _blk(HEADS * BASES), _blk(BASES * F),
                  _blk(BASES * F), _full((1, HID)),
                  _full((BASES, HEADS * BASES, HID)),
                  _full((BASES, BASES * F, HID))],
        out_specs=[_blk(HID), pl.BlockSpec((1, 1, HID), lambda i: (i, 0, 0)),
                   pl.BlockSpec((1, 1, HID), lambda i: (i, 0, 0))],
        out_shape=[
            jax.ShapeDtypeStruct((N, HID), jnp.float32),
            jax.ShapeDtypeStruct((NBLK, 1, HID), jnp.float32),
            jax.ShapeDtypeStruct((NBLK, 1, HID), jnp.float32),
        ],
    )(dinv, wt, p0, p1, bias, s_e, t_e)


def _tc_tail_b(h, o, ps, pss, g, be, dinv, wb, wc, bc):
    def body(h_r, o_r, ps_r, pss_r, g_r, be_r, dinv_r, wb_r, wc_r, bc_r,
             hn_r, bases_r, wt_r):
        mu, rstd = _finalize_stats(ps_r, pss_r)
        hn = h_r[...] + jax.nn.relu((o_r[...] - mu) * rstd * g_r[...]
                                    + be_r[...])
        hn_r[...] = hn
        bases_r[...] = dinv_r[...] * _dot(hn, wb_r[...])
        wt_r[...] = _dot(hn, wc_r[...]) + bc_r[...]

    return pl.pallas_call(
        body,
        grid=(NBLK,),
        in_specs=[_blk(HID), _blk(HID), _full((NBLK, 1, HID)),
                  _full((NBLK, 1, HID)), _full((1, HID)), _full((1, HID)),
                  _blk(1), _full((HID, BASES * F)),
                  _full((HID, HEADS * BASES)), _full((1, HEADS * BASES))],
        out_specs=[_blk(HID), _blk(BASES * F), _blk(HEADS * BASES)],
        out_shape=[
            jax.ShapeDtypeStruct((N, HID), jnp.float32),
            jax.ShapeDtypeStruct((N, BASES * F), jnp.float32),
            jax.ShapeDtypeStruct((N, HEADS * BASES), jnp.float32),
        ],
    )(h, o, ps, pss, g, be, dinv, wb, wc, bc)


def _tc_pool(hn, batch_col):
    def body(h_r, batch_r, ssum_r, cnt_r):
        i = pl.program_id(0)
        gids = jax.lax.broadcasted_iota(jnp.int32, (1, NGRAPH), 1)
        onehot = (batch_r[...] == gids).astype(jnp.float32)

        @pl.when(i == 0)
        def _():
            ssum_r[...] = jnp.zeros((NGRAPH, HID), jnp.float32)
            cnt_r[...] = jnp.zeros((NGRAPH, 1), jnp.float32)

        ssum_r[...] += _dotT(onehot, h_r[...])
        cnt_r[...] += _dotT(onehot, jnp.ones((RB, 1), jnp.float32))

    return pl.pallas_call(
        body,
        grid=(NBLK,),
        in_specs=[_blk(HID), _blk(1)],
        out_specs=[_full((NGRAPH, HID)), _full((NGRAPH, 1))],
        out_shape=[
            jax.ShapeDtypeStruct((NGRAPH, HID), jnp.float32),
            jax.ShapeDtypeStruct((NGRAPH, 1), jnp.float32),
        ],
    )(hn, batch_col)


def _tc_head(ssum, cnt, descriptors, Wm1, gm1, bm1, Wm2, gm2, bm2, W2m,
             W2d, b2, g3, be3, Wout, bout):
    def body(ssum_r, cnt_r, desc_r, wm1_r, gm1_r, bm1_r, wm2_r, gm2_r,
             bm2_r, w2m_r, w2d_r, b2_r, g3_r, be3_r, wout_r, bout_r, out_r):
        pooled = ssum_r[...] * (1.0 / jnp.maximum(cnt_r[...], 1.0))
        m = jax.nn.relu(_bn(_dot(pooled, wm1_r[...]), gm1_r[...], bm1_r[...]))
        m = jax.nn.relu(_bn(_dot(m, wm2_r[...]), gm2_r[...], bm2_r[...]))
        z = jax.nn.relu(_dot(m, w2m_r[...]) + _dot(desc_r[...], w2d_r[...])
                        + b2_r[...])
        z = _bn(z, g3_r[...], be3_r[...])
        out_r[...] = _dot(z, wout_r[...]) + bout_r[...]

    return pl.pallas_call(
        body,
        out_shape=jax.ShapeDtypeStruct((NGRAPH, 1), jnp.float32),
    )(ssum, cnt, descriptors, Wm1, gm1, bm1, Wm2, gm2, bm2, W2m, W2d,
      b2, g3, be3, Wout, bout)


def kernel(x, edge_index, batch, descriptors, W1, b1, g1, be1, convWb, convWc,
           convbc, convbias, convg, convbe, Wm1, gm1, bm1, Wm2, gm2, bm2, W2,
           b2, g3, be3, Wout, bout):
    n = N
    E = edge_index.shape[1]
    epd = ((E + NW * CH - 1) // (NW * CH)) * (NW * CH)
    dst3d = _pad_edges(edge_index[1], epd).reshape(NW, epd // (NW * CH), CH)
    parts = _sc_degree_kernel(epd // (NW * CH))(dst3d)
    deg = 1.0 + parts[0, :N] + parts[1, :N]
    dinv = lax.rsqrt(deg)

    loop = jnp.arange(n, dtype=edge_index.dtype)
    etot = E + n
    ep = ((etot + 3 * NW * CH - 1) // (3 * NW * CH)) * (3 * NW * CH)
    npad_e = ep - etot
    src_pad = jnp.asarray((np.arange(npad_e) * 61) % N, jnp.int32)
    dst_pad = jnp.asarray(N + (np.arange(npad_e) % (NPAD - N)), jnp.int32)
    epw_chunks = ep // (NW * CH)
    src3d = jnp.concatenate([edge_index[0], loop, src_pad]).reshape(NW, epw_chunks, CH)
    dst3d = jnp.concatenate([edge_index[1], loop, dst_pad]).reshape(NW, epw_chunks, CH)
    layer_scatter = _sc_layer_kernel(epw_chunks)

    xp = jnp.pad(x, ((0, 0), (0, 5)))
    w1p = jnp.pad(W1, ((0, 5), (0, 0)))
    s_e = jnp.asarray(_S_EXPAND)
    t_e = jnp.asarray(_T_EXPAND)
    d0 = parts[0, :N, None]
    d1 = parts[1, :N, None]
    t0, eps_, epss_ = _tc_embed_a(xp, w1p, b1[None, :])
    h, dinv2, bases, wt = _tc_embed_b(
        t0, eps_, epss_, g1[None, :], be1[None, :], d0, d1,
        convWb[0], convWc[0], convbc[0][None, :])
    for l in range(LAYERS):
        ps = layer_scatter(src3d, dst3d, bases)
        o, ops_, opss_ = _tc_tail_a(
            dinv2, wt, ps[0, :N], ps[1, :N], convbias[l][None, :], s_e, t_e)
        ln = min(l + 1, LAYERS - 1)
        h, bases, wt = _tc_tail_b(
            h, o, ops_, opss_, convg[l][None, :], convbe[l][None, :], dinv2,
            convWb[ln], convWc[ln], convbc[ln][None, :])
    ssum, cnt = _tc_pool(h, batch[:, None])
    return _tc_head(
        ssum, cnt, descriptors, Wm1, gm1[None, :], bm1[None, :],
        Wm2, gm2[None, :], bm2[None, :], W2[:HID // 4], W2[HID // 4:],
        b2[None, :], g3[None, :], be3[None, :], Wout, bout[None, :])
